# Initial kernel scaffold; baseline (speedup 1.0000x reference)
import jax
import jax.numpy as jnp
from jax.experimental import pallas as pl


def kernel(x, xe, pair, a, wt):
    raise NotImplementedError("write your pallas kernel here")


# speedup vs baseline: 5.0397x; 5.0397x over previous
import functools

import jax
import jax.numpy as jnp
from jax import lax
from jax.experimental import pallas as pl
from jax.experimental.pallas import tpu as pltpu
from jax.experimental.pallas import tpu_sc as plsc

N_NODE = 10000
N_EDGE = 2000
N_PAIR = 160000
D = 128

NC = 2
NS = 16
NPASS = 2
ROWS_PER_PASS = N_EDGE // (NC * NPASS)
PASS_WORDS = ROWS_PER_PASS * N_EDGE
PAD = 8
CHUNK = N_PAIR // NS
VREGS = CHUNK // 16
IDX_ROWS = (CHUNK + 127) // 128
ZCHUNK = 8000
NZCHUNK = PASS_WORDS // ZCHUNK


def _mask_body(p0_hbm, p1_hbm, out_hbm, shared, p0v, p1v, idxv, onesv, zbuf, obuf):
    cid = lax.axis_index("c")
    sid = lax.axis_index("s")

    def _fill_z(i, _):
        zbuf[pl.ds(i * 16, 16)] = jnp.zeros((16,), jnp.float32)
        return 0

    lax.fori_loop(0, ZCHUNK // 16, _fill_z, 0)
    for k in range(8):
        onesv[pl.ds(k * 16, 16)] = jnp.ones((16,), jnp.float32)

    pltpu.sync_copy(p0_hbm.at[pl.ds(sid * CHUNK, CHUNK)], p0v)
    pltpu.sync_copy(p1_hbm.at[pl.ds(sid * CHUNK, CHUNK)], p1v)

    for ps in range(NPASS):
        row_base = (cid * NPASS + ps) * ROWS_PER_PASS

        for k in range(-(-NZCHUNK // NS)):
            t = sid + NS * k
            if NZCHUNK % NS == 0 or k < NZCHUNK // NS:
                pltpu.sync_copy(zbuf, shared.at[pl.ds(t * ZCHUNK, ZCHUNK)])
            else:
                @pl.when(t < NZCHUNK)
                def _():
                    pltpu.sync_copy(zbuf, shared.at[pl.ds(t * ZCHUNK, ZCHUNK)])

        def _offsets(i, _):
            p0 = p0v[pl.ds(i * 16, 16)]
            p1 = p1v[pl.ds(i * 16, 16)]
            r = p0 - row_base
            ok = (r >= 0) & (r < ROWS_PER_PASS)
            off = jnp.where(ok, r * N_EDGE + p1, PASS_WORDS)
            idxv[i // 8, pl.ds((i % 8) * 16, 16)] = off
            return 0

        lax.fori_loop(0, VREGS, _offsets, 0)
        for k in range(CHUNK % 128 // 16, 8):
            idxv[IDX_ROWS - 1, pl.ds(k * 16, 16)] = jnp.full(
                (16,), PASS_WORDS, jnp.int32)

        plsc.subcore_barrier()

        for j in range(IDX_ROWS):
            pltpu.sync_copy(onesv, shared.at[idxv.at[j]], add=True)

        plsc.subcore_barrier()

        out_base = row_base * N_EDGE
        for k in range(-(-NZCHUNK // NS)):
            t = sid + NS * k

            def _copy_out(t=t):
                pltpu.sync_copy(shared.at[pl.ds(t * ZCHUNK, ZCHUNK)], obuf)
                pltpu.sync_copy(
                    obuf, out_hbm.at[pl.ds(out_base + t * ZCHUNK, ZCHUNK)])

            if NZCHUNK % NS == 0 or k < NZCHUNK // NS:
                _copy_out()
            else:
                pl.when(t < NZCHUNK)(_copy_out)

        if ps + 1 < NPASS:
            plsc.subcore_barrier()


_build_mask = functools.partial(
    pl.kernel,
    mesh=plsc.VectorSubcoreMesh(core_axis_name="c", subcore_axis_name="s"),
    out_type=jax.ShapeDtypeStruct((N_EDGE * N_EDGE,), jnp.float32),
    scratch_types=[
        pltpu.VMEM_SHARED((PASS_WORDS + PAD,), jnp.float32),
        pltpu.VMEM((CHUNK,), jnp.int32),
        pltpu.VMEM((CHUNK,), jnp.int32),
        pltpu.VMEM((IDX_ROWS, 128), jnp.int32),
        pltpu.VMEM((128,), jnp.float32),
        pltpu.VMEM((ZCHUNK,), jnp.float32),
        pltpu.VMEM((ZCHUNK,), jnp.float32),
    ],
)(_mask_body)


def _proj_body(x_ref, xe_ref, wt_ref, a_ref, xep_ref, xa_ref, xph_ref, ts_ref):
    wt = wt_ref[...]
    xp = jnp.dot(x_ref[...], wt, preferred_element_type=jnp.float32)
    xep_ref[...] = jnp.dot(xe_ref[...], wt, preferred_element_type=jnp.float32)
    xph = xp[:N_EDGE]
    xph_ref[...] = xph
    xa_ref[...] = xph * a_ref[...]
    ts_ref[...] = jnp.sum(xp[N_EDGE:], axis=0, keepdims=True)


_project = pl.pallas_call(
    _proj_body,
    out_shape=[
        jax.ShapeDtypeStruct((N_EDGE, D), jnp.float32),
        jax.ShapeDtypeStruct((N_EDGE, D), jnp.float32),
        jax.ShapeDtypeStruct((N_EDGE, D), jnp.float32),
        jax.ShapeDtypeStruct((1, D), jnp.float32),
    ],
)


def _attn_body(xep_ref, xa_ref, xph_ref, ts_ref, m_ref, xout_ref, xeout_ref):
    xep = xep_ref[...]
    s = lax.dot_general(xep, xa_ref[...], (((1,), (1,)), ((), ())),
                        preferred_element_type=jnp.float32)
    g = 1e-10 + jnp.where(m_ref[...] > 0, jnp.exp(s), 0.0)

    ze = jnp.sum(g, axis=1, keepdims=True) + (N_NODE - N_EDGE) * 1e-10
    pe = g / ze
    xe_out = (jnp.dot(pe, xph_ref[...], preferred_element_type=jnp.float32)
              + (1e-10 / ze) * ts_ref[...])
    xeout_ref[...] = jnp.where(xe_out > 0, xe_out, jnp.exp(xe_out) - 1.0)

    zn = jnp.sum(g, axis=0, keepdims=True)
    qn = g / zn
    x_head = lax.dot_general(qn, xep, (((0,), (0,)), ((), ())),
                             preferred_element_type=jnp.float32)
    xout_ref[:N_EDGE] = jnp.where(x_head > 0, x_head, jnp.exp(x_head) - 1.0)
    x_tail = jnp.sum(xep, axis=0, keepdims=True) * (1.0 / N_EDGE)
    x_tail = jnp.where(x_tail > 0, x_tail, jnp.exp(x_tail) - 1.0)
    xout_ref[N_EDGE:] = jnp.broadcast_to(x_tail, (N_NODE - N_EDGE, D))


_attend = pl.pallas_call(
    _attn_body,
    out_shape=[
        jax.ShapeDtypeStruct((N_NODE, D), jnp.float32),
        jax.ShapeDtypeStruct((N_EDGE, D), jnp.float32),
    ],
)


def kernel(x, xe, pair, a, wt):
    m = _build_mask(pair[0], pair[1]).reshape(N_EDGE, N_EDGE)
    xep, xa, xph, ts = _project(x, xe, wt, a.reshape(1, D))
    x_out, xe_out = _attend(xep, xa, xph, ts, m)
    return x_out, xe_out
